# baseline (speedup 1.0000x reference)
import functools

import jax
import jax.numpy as jnp
from jax import lax
from jax.experimental import pallas as pl
from jax.experimental.pallas import tpu as pltpu
from jax.experimental.pallas import tpu_sc as plsc

NUM_FEATURES = 4096
BATCH = 16384

_info = plsc.get_sparse_core_info()
_NC, _NS, _L = _info.num_cores, _info.num_subcores, _info.num_lanes
_NW = _NC * _NS
_ROWS_PER_W = BATCH // _NW
_R = 8
_CHUNKS = _ROWS_PER_W // _R
_HALF = NUM_FEATURES // 2
_KH = _HALF // _L


def _permute_sc(x, perm32):
    mesh = plsc.VectorSubcoreMesh(core_axis_name="c", subcore_axis_name="s")

    @functools.partial(
        pl.kernel,
        mesh=mesh,
        out_type=jax.ShapeDtypeStruct((BATCH, NUM_FEATURES), jnp.float32),
        compiler_params=pltpu.CompilerParams(needs_layout_passes=False),
        scratch_types=[
            pltpu.VMEM((NUM_FEATURES,), jnp.int32),
            pltpu.VMEM((_R, NUM_FEATURES), jnp.float32),
            pltpu.VMEM((_R, NUM_FEATURES), jnp.float32),
            pltpu.VMEM((_R, _HALF), jnp.float32),
            pltpu.VMEM((_R, _HALF), jnp.float32),
            pltpu.SemaphoreType.DMA,
            pltpu.SemaphoreType.DMA,
            pltpu.SemaphoreType.DMA,
            pltpu.SemaphoreType.DMA,
        ],
    )
    def permute(x_hbm, perm_hbm, out_hbm, perm_v, in0, in1, outh0, outh1,
                isem0, isem1, osem0, osem1):
        wid = lax.axis_index("s") * _NC + lax.axis_index("c")
        pltpu.sync_copy(perm_hbm, perm_v)
        base0 = wid * _ROWS_PER_W
        ins, isems = (in0, in1), (isem0, isem1)
        outs, osems = (outh0, outh1), (osem0, osem1)
        rvecs = [jnp.full((_L,), r, jnp.int32) for r in range(_R)]

        def in_copy(c, b):
            return pltpu.make_async_copy(
                x_hbm.at[pl.ds(base0 + c * _R, _R)], ins[b], isems[b])

        def out_copy(c, h):
            return pltpu.make_async_copy(
                outs[h],
                out_hbm.at[pl.ds(base0 + c * _R, _R), pl.ds(h * _HALF, _HALF)],
                osems[h])

        in_copy(0, 0).start()
        in_copy(1, 1).start()

        @pl.loop(0, _CHUNKS, step=2)
        def chunk_loop(g):
            for b in range(2):
                c = g + b
                in_copy(c, b).wait()
                inb = ins[b]

                for h in range(2):
                    @pl.when(c >= 1)
                    def _():
                        out_copy(c - 1, h).wait()

                    outb = outs[h]

                    @plsc.parallel_loop(0, _KH, unroll=8)
                    def kbody(kk):
                        idx = perm_v[pl.ds((h * _KH + kk) * _L, _L)]
                        for r in range(_R):
                            val = plsc.load_gather(inb, [rvecs[r], idx])
                            outb[r, pl.ds(kk * _L, _L)] = val

                    out_copy(c, h).start()

                @pl.when(c + 2 < _CHUNKS)
                def _():
                    in_copy(c + 2, b).start()

        out_copy(_CHUNKS - 1, 0).wait()
        out_copy(_CHUNKS - 1, 1).wait()

    return permute(x, perm32)


def kernel(x, perm, inv_perm):
    del inv_perm
    return _permute_sc(x, perm.astype(jnp.int32))


# speedup vs baseline: 1.0219x; 1.0219x over previous
import functools

import jax
import jax.numpy as jnp
from jax import lax
from jax.experimental import pallas as pl
from jax.experimental.pallas import tpu as pltpu
from jax.experimental.pallas import tpu_sc as plsc

NUM_FEATURES = 4096
BATCH = 16384

_info = plsc.get_sparse_core_info()
_NC, _NS, _L = _info.num_cores, _info.num_subcores, _info.num_lanes
_NW = _NC * _NS
_ROWS_PER_W = BATCH // _NW
_R = 8
_CHUNKS = _ROWS_PER_W // _R
_SPLIT = 2432
_REST = NUM_FEATURES - _SPLIT
_KS = _SPLIT // _L
_KR = _REST // _L


def _permute_sc(x, perm32):
    mesh = plsc.VectorSubcoreMesh(core_axis_name="c", subcore_axis_name="s")

    @functools.partial(
        pl.kernel,
        mesh=mesh,
        out_type=jax.ShapeDtypeStruct((BATCH, NUM_FEATURES), jnp.float32),
        compiler_params=pltpu.CompilerParams(needs_layout_passes=False),
        scratch_types=[
            pltpu.VMEM((NUM_FEATURES,), jnp.int32),
            pltpu.VMEM((_R, NUM_FEATURES), jnp.float32),
            pltpu.VMEM((_R, NUM_FEATURES), jnp.float32),
            pltpu.VMEM((_R, _SPLIT), jnp.float32),
            pltpu.VMEM((_R, _REST), jnp.float32),
            pltpu.VMEM_SHARED((_NS, 2, _R, _REST), jnp.float32),
            pltpu.SemaphoreType.DMA,
            pltpu.SemaphoreType.DMA,
            pltpu.SemaphoreType.DMA,
            pltpu.SemaphoreType.DMA,
            pltpu.SemaphoreType.DMA,
            pltpu.SemaphoreType.DMA,
            pltpu.SemaphoreType.DMA,
        ],
    )
    def permute(x_hbm, perm_hbm, out_hbm, perm_v, in0, in1, sout,
                spbuf, shared, isem0, isem1, osem0, osem1, tsem, hsem0, hsem1):
        wid = lax.axis_index("s") * _NC + lax.axis_index("c")
        sid = lax.axis_index("s")
        pltpu.sync_copy(perm_hbm, perm_v)
        base0 = wid * _ROWS_PER_W
        ins, isems = (in0, in1), (isem0, isem1)
        osems = (osem0, osem1)
        hsems = (hsem0, hsem1)
        rvecs = [jnp.full((_L,), r, jnp.int32) for r in range(_R)]

        def in_copy(c, b):
            return pltpu.make_async_copy(
                x_hbm.at[pl.ds(base0 + c * _R, _R)], ins[b], isems[b])

        def out_copy(c, b):
            return pltpu.make_async_copy(
                sout,
                out_hbm.at[pl.ds(base0 + c * _R, _R), pl.ds(0, _SPLIT)],
                osems[b])

        def t2s(b):
            return pltpu.make_async_copy(spbuf, shared.at[sid, b], tsem)

        def s2h(c, b):
            return pltpu.make_async_copy(
                shared.at[sid, b],
                out_hbm.at[pl.ds(base0 + c * _R, _R), pl.ds(_SPLIT, _REST)],
                hsems[b])

        in_copy(0, 0).start()
        in_copy(1, 1).start()

        @pl.loop(0, _CHUNKS, step=2)
        def chunk_loop(g):
            for b in range(2):
                c = g + b
                in_copy(c, b).wait()
                inb = ins[b]

                @pl.when(c >= 1)
                def _():
                    out_copy(c - 1, 1 - b).wait()

                @plsc.parallel_loop(0, _KS, unroll=8)
                def kbody_s(kk):
                    idx = perm_v[pl.ds(kk * _L, _L)]
                    for r in range(_R):
                        val = plsc.load_gather(inb, [rvecs[r], idx])
                        sout[r, pl.ds(kk * _L, _L)] = val

                out_copy(c, b).start()

                @pl.when(c >= 2)
                def _():
                    s2h(c - 2, b).wait()

                @plsc.parallel_loop(0, _KR, unroll=8)
                def kbody_r(kk):
                    idx = perm_v[pl.ds(_SPLIT + kk * _L, _L)]
                    for r in range(_R):
                        val = plsc.load_gather(inb, [rvecs[r], idx])
                        spbuf[r, pl.ds(kk * _L, _L)] = val

                @pl.when(c + 2 < _CHUNKS)
                def _():
                    in_copy(c + 2, b).start()

                t2s(b).start()
                t2s(b).wait()
                s2h(c, b).start()

        out_copy(_CHUNKS - 1, 1).wait()
        s2h(_CHUNKS - 2, 0).wait()
        s2h(_CHUNKS - 1, 1).wait()

    return permute(x, perm32)


def kernel(x, perm, inv_perm):
    del inv_perm
    return _permute_sc(x, perm.astype(jnp.int32))
